# Initial kernel scaffold; baseline (speedup 1.0000x reference)
import functools

import jax
import jax.numpy as jnp
from jax import lax
from jax.experimental import pallas as pl
from jax.experimental.pallas import tpu as pltpu
from jax.experimental.pallas import tpu_sc as plsc

_SCALE = 0.1
_K = 8
_L = 5
_P = 1024
_D = 768
_B = 128
_S = 196
_OUT_S = _K * _L + _S
_HEAD = _K * _L
_ROW_W = _L * _D

_NC = 2
_NS = 16
_NW = _NC * _NS
_ROWS_PER_W = _B // _NW
_CHUNKS = _P // 16


_BT = 8


def _mean_copy_body(x_ref, big_ref, mean_ref):
    xb = x_ref[...]
    big_ref[:, :_HEAD, :] = jnp.zeros((_BT, _HEAD, _D), jnp.float32)
    big_ref[:, _HEAD:, :] = xb
    mean_ref[...] = jnp.mean(xb, axis=1, keepdims=True)


def _run_mean_copy(x_embed):
    return pl.pallas_call(
        _mean_copy_body,
        grid=(_B // _BT,),
        in_specs=[pl.BlockSpec((_BT, _S, _D), lambda i: (i, 0, 0))],
        out_specs=[
            pl.BlockSpec((_BT, _OUT_S, _D), lambda i: (i, 0, 0)),
            pl.BlockSpec((_BT, 1, _D), lambda i: (i, 0, 0)),
        ],
        out_shape=[
            jax.ShapeDtypeStruct((_B, _OUT_S, _D), jnp.float32),
            jax.ShapeDtypeStruct((_B, 1, _D), jnp.float32),
        ],
    )(x_embed)


def _map_to_ball(u):
    ss = jnp.sum(u * u, axis=-1, keepdims=True)
    un = u * lax.rsqrt(jnp.maximum(ss, 1e-12))
    us = un * _SCALE
    n2 = jnp.sum(us * us, axis=-1, keepdims=True)
    n = jnp.sqrt(jnp.maximum(n2, 1e-15))
    y = jnp.tanh(n) * us / n
    yn2 = jnp.sum(y * y, axis=-1, keepdims=True)
    ynorm = jnp.sqrt(jnp.maximum(yn2, 1e-15))
    maxnorm = 1.0 - 4e-3
    return jnp.where(ynorm > maxnorm, y / ynorm * maxnorm, y)


def _dist_body(mean_ref, pk_ref, sim_ref, yn_ref):
    x = _map_to_ball(mean_ref[...])
    y = _map_to_ball(pk_ref[...])
    yn_ref[...] = y
    x2 = jnp.sum(x * x, axis=-1, keepdims=True)
    y2 = jnp.sum(y * y, axis=-1)[None, :]
    xy = lax.dot_general(x, y, (((1,), (1,)), ((), ())),
                         precision=lax.Precision.HIGHEST,
                         preferred_element_type=jnp.float32)
    alpha = 1.0 - 2.0 * xy + y2
    beta = 1.0 - x2
    num2 = alpha * alpha * x2 + beta * beta * y2 - 2.0 * alpha * beta * xy
    den = 1.0 - 2.0 * xy + x2 * y2
    norm = jnp.sqrt(jnp.maximum(num2, 1e-15)) / (den + 1e-15)
    arg = jnp.clip(norm, 0.0, 1.0 - 1e-7)
    dist = jnp.log((1.0 + arg) / (1.0 - arg))
    sim_ref[...] = -dist


def _run_dist(mean2, prompt_key):
    return pl.pallas_call(
        _dist_body,
        out_shape=[
            jax.ShapeDtypeStruct((_B, _P), jnp.float32),
            jax.ShapeDtypeStruct((_P, _D), jnp.float32),
        ],
    )(mean2, prompt_key)


def _sc_body(sim_hbm, prompt_hbm, key_hbm,
             head_hbm, idx_hbm, bkn_hbm, part_hbm,
             sim_v, idx16_v, idx8_v, rows_v, bkn_v, sum_v, sem, sem2):
    cid = lax.axis_index("c")
    sid = lax.axis_index("s")
    wid = cid * _NS + sid
    lane = lax.iota(jnp.int32, 16)
    neg = jnp.full((16,), -3.0e38, jnp.float32)
    zidx = jnp.zeros((16,), jnp.int32)

    def row_body(r, acc):
        b = wid * _ROWS_PER_W + r
        pltpu.sync_copy(sim_hbm.at[b], sim_v)

        def chunk_body(i, carry):
            bk, bi = carry
            ck = sim_v[pl.ds(i * 16, 16)]
            ci = lane + i * 16
            sck, sci = plsc.sort_key_val(ck, ci, descending=True)
            rk = lax.rev(sck, (0,))
            ri = lax.rev(sci, (0,))
            mk = jnp.where(lane < 8, bk, rk)
            mi = jnp.where(lane < 8, bi, ri)
            return plsc.sort_key_val(mk, mi, descending=True)

        bk, bi = lax.fori_loop(0, _CHUNKS, chunk_body, (neg, zidx))
        iv = jnp.where(lane < 8, bi, jnp.int32(2147483647))
        fi, fv = plsc.sort_key_val(iv, bk, descending=False)
        idx16_v[...] = fi
        plsc.store_compressed(idx8_v, fi, mask=lane < 8)
        pltpu.sync_copy(idx16_v.at[pl.ds(0, 8)], idx_hbm.at[b])
        pltpu.async_copy(prompt_hbm.at[idx8_v], rows_v, sem).wait()
        pltpu.sync_copy(rows_v, head_hbm.at[b])
        pltpu.async_copy(key_hbm.at[idx8_v], bkn_v, sem2).wait()
        pltpu.sync_copy(bkn_v, bkn_hbm.at[b])
        sel = jnp.where(lane < 8, fv, jnp.float32(0.0))
        return acc - jnp.sum(sel)

    acc = lax.fori_loop(0, _ROWS_PER_W, row_body, jnp.float32(0.0))
    sv = jnp.where(lane == 0, jnp.full((16,), acc), jnp.zeros((16,), jnp.float32))
    sum_v[...] = sv
    pltpu.sync_copy(sum_v, part_hbm.at[wid])


def _run_topk_gather(sim, prompt2, yn):
    mesh = plsc.VectorSubcoreMesh(core_axis_name="c", subcore_axis_name="s",
                                  num_cores=_NC, num_subcores=_NS)
    fn = pl.kernel(
        _sc_body,
        out_type=[
            jax.ShapeDtypeStruct((_B, _K, _ROW_W), jnp.float32),
            jax.ShapeDtypeStruct((_B, _K), jnp.int32),
            jax.ShapeDtypeStruct((_B, _K, _D), jnp.float32),
            jax.ShapeDtypeStruct((_NW, 16), jnp.float32),
        ],
        mesh=mesh,
        scratch_types=[
            pltpu.VMEM((_P,), jnp.float32),
            pltpu.VMEM((16,), jnp.int32),
            pltpu.VMEM((8,), jnp.int32),
            pltpu.VMEM((_K, _ROW_W), jnp.float32),
            pltpu.VMEM((_K, _D), jnp.float32),
            pltpu.VMEM((16,), jnp.float32),
            pltpu.SemaphoreType.DMA,
            pltpu.SemaphoreType.DMA,
        ],
    )
    return fn(sim, prompt2, yn)


def _assemble_body(big_in_ref, head_ref, part_ref, big_ref, rs_ref):
    del big_in_ref
    big_ref[...] = head_ref[...]

    @pl.when(pl.program_id(0) == 0)
    def _():
        rs_ref[0, 0] = jnp.sum(part_ref[...]) / float(_B)


def _run_assemble(big0, head3, part):
    return pl.pallas_call(
        _assemble_body,
        grid=(_B,),
        in_specs=[
            pl.BlockSpec(memory_space=pltpu.ANY),
            pl.BlockSpec((1, _HEAD, _D), lambda b: (b, 0, 0)),
            pl.BlockSpec((_NW, 16), lambda b: (0, 0)),
        ],
        out_specs=[
            pl.BlockSpec((1, _HEAD, _D), lambda b: (b, 0, 0)),
            pl.BlockSpec((1, 1), lambda b: (0, 0)),
        ],
        out_shape=[
            jax.ShapeDtypeStruct((_B, _OUT_S, _D), jnp.float32),
            jax.ShapeDtypeStruct((1, 1), jnp.float32),
        ],
        input_output_aliases={0: 0},
    )(big0, head3, part)


def kernel(x_embed, prompt, prompt_key):
    big0, mean3 = _run_mean_copy(x_embed)
    sim, yn = _run_dist(mean3.reshape(_B, _D), prompt_key)
    head, idx, bkn, part = _run_topk_gather(
        sim, prompt.reshape(_P, _ROW_W), yn)
    big, rs = _run_assemble(big0, head.reshape(_B, _HEAD, _D), part)
    return big, rs[0, 0], sim, idx, bkn


# speedup vs baseline: 1.2249x; 1.2249x over previous
import functools

import jax
import jax.numpy as jnp
from jax import lax
from jax.experimental import pallas as pl
from jax.experimental.pallas import tpu as pltpu
from jax.experimental.pallas import tpu_sc as plsc

_SCALE = 0.1
_K = 8
_L = 5
_P = 1024
_D = 768
_B = 128
_S = 196
_OUT_S = _K * _L + _S
_HEAD = _K * _L
_ROW_W = _L * _D

_NC = 2
_NS = 16
_NW = _NC * _NS
_ROWS_PER_W = _B // _NW
_CHUNKS = _P // 16


_BT = 8


def _mean_copy_body(x_ref, big_ref, mean_ref):
    xb = x_ref[...]
    big_ref[:, :_HEAD, :] = jnp.zeros((_BT, _HEAD, _D), jnp.float32)
    big_ref[:, _HEAD:, :] = xb
    mean_ref[...] = jnp.mean(xb, axis=1, keepdims=True)


def _run_mean_copy(x_embed):
    return pl.pallas_call(
        _mean_copy_body,
        grid=(_B // _BT,),
        in_specs=[pl.BlockSpec((_BT, _S, _D), lambda i: (i, 0, 0))],
        out_specs=[
            pl.BlockSpec((_BT, _OUT_S, _D), lambda i: (i, 0, 0)),
            pl.BlockSpec((_BT, 1, _D), lambda i: (i, 0, 0)),
        ],
        out_shape=[
            jax.ShapeDtypeStruct((_B, _OUT_S, _D), jnp.float32),
            jax.ShapeDtypeStruct((_B, 1, _D), jnp.float32),
        ],
    )(x_embed)


def _map_to_ball(u):
    ss = jnp.sum(u * u, axis=-1, keepdims=True)
    un = u * lax.rsqrt(jnp.maximum(ss, 1e-12))
    us = un * _SCALE
    n2 = jnp.sum(us * us, axis=-1, keepdims=True)
    n = jnp.sqrt(jnp.maximum(n2, 1e-15))
    y = jnp.tanh(n) * us / n
    yn2 = jnp.sum(y * y, axis=-1, keepdims=True)
    ynorm = jnp.sqrt(jnp.maximum(yn2, 1e-15))
    maxnorm = 1.0 - 4e-3
    return jnp.where(ynorm > maxnorm, y / ynorm * maxnorm, y)


def _dist_body(mean_ref, pk_ref, sim_ref, yn_ref):
    x = _map_to_ball(mean_ref[...])
    y = _map_to_ball(pk_ref[...])
    yn_ref[...] = y
    x2 = jnp.sum(x * x, axis=-1, keepdims=True)
    y2 = jnp.sum(y * y, axis=-1)[None, :]
    xy = lax.dot_general(x, y, (((1,), (1,)), ((), ())),
                         precision=lax.Precision.HIGHEST,
                         preferred_element_type=jnp.float32)
    alpha = 1.0 - 2.0 * xy + y2
    beta = 1.0 - x2
    num2 = alpha * alpha * x2 + beta * beta * y2 - 2.0 * alpha * beta * xy
    den = 1.0 - 2.0 * xy + x2 * y2
    norm = jnp.sqrt(jnp.maximum(num2, 1e-15)) / (den + 1e-15)
    arg = jnp.clip(norm, 0.0, 1.0 - 1e-7)
    dist = jnp.log((1.0 + arg) / (1.0 - arg))
    sim_ref[...] = -dist


def _run_dist(mean2, prompt_key):
    return pl.pallas_call(
        _dist_body,
        out_shape=[
            jax.ShapeDtypeStruct((_B, _P), jnp.float32),
            jax.ShapeDtypeStruct((_P, _D), jnp.float32),
        ],
    )(mean2, prompt_key)


def _sc_body(sim_hbm, prompt_hbm, key_hbm,
             head_hbm, idx_hbm, bkn_hbm, part_hbm,
             sim_v, idx16_v, rows_v, bkn_v, sum_v, sem, sem2):
    cid = lax.axis_index("c")
    sid = lax.axis_index("s")
    wid = cid * _NS + sid
    lane = lax.iota(jnp.int32, 16)
    neg = jnp.full((16,), -3.0e38, jnp.float32)
    zidx = jnp.zeros((16,), jnp.int32)

    def row_body(r, acc):
        b = wid * _ROWS_PER_W + r
        pltpu.sync_copy(sim_hbm.at[b], sim_v)

        def chunk_body(i, carry):
            bk, bi = carry
            ck = sim_v[pl.ds(i * 16, 16)]
            ci = lane + i * 16
            sck, sci = plsc.sort_key_val(ck, ci, descending=True)
            rk = lax.rev(sck, (0,))
            ri = lax.rev(sci, (0,))
            mk = jnp.where(lane < 8, bk, rk)
            mi = jnp.where(lane < 8, bi, ri)
            return tuple(plsc.sort_key_val(mk, mi, descending=True))

        bk, bi = lax.fori_loop(0, _CHUNKS, chunk_body, (neg, zidx))
        iv = jnp.where(lane < 8, bi, jnp.int32(2147483647))
        fi, fv = plsc.sort_key_val(iv, bk, descending=False)
        idx16_v[...] = fi
        idx8 = idx16_v.at[pl.ds(0, 8)]
        pltpu.sync_copy(idx8, idx_hbm.at[b])
        pltpu.async_copy(prompt_hbm.at[idx8], rows_v, sem).wait()
        pltpu.sync_copy(rows_v, head_hbm.at[b])
        pltpu.async_copy(key_hbm.at[idx8], bkn_v, sem2).wait()
        pltpu.sync_copy(bkn_v, bkn_hbm.at[b])
        sel = jnp.where(lane < 8, fv, jnp.float32(0.0))
        return acc - jnp.sum(sel)

    acc = lax.fori_loop(0, _ROWS_PER_W, row_body, jnp.float32(0.0))
    sv = jnp.where(lane == 0, jnp.full((16,), acc), jnp.zeros((16,), jnp.float32))
    sum_v[...] = sv
    pltpu.sync_copy(sum_v, part_hbm.at[wid])


def _run_topk_gather(sim, prompt2, yn):
    mesh = plsc.VectorSubcoreMesh(core_axis_name="c", subcore_axis_name="s",
                                  num_cores=_NC, num_subcores=_NS)
    fn = pl.kernel(
        _sc_body,
        out_type=[
            jax.ShapeDtypeStruct((_B, _K, _ROW_W), jnp.float32),
            jax.ShapeDtypeStruct((_B, _K), jnp.int32),
            jax.ShapeDtypeStruct((_B, _K, _D), jnp.float32),
            jax.ShapeDtypeStruct((_NW, 16), jnp.float32),
        ],
        mesh=mesh,
        compiler_params=pltpu.CompilerParams(needs_layout_passes=False,
                                             use_tc_tiling_on_sc=False),
        scratch_types=[
            pltpu.VMEM((_P,), jnp.float32),
            pltpu.VMEM((16,), jnp.int32),
            pltpu.VMEM((_K, _ROW_W), jnp.float32),
            pltpu.VMEM((_K, _D), jnp.float32),
            pltpu.VMEM((16,), jnp.float32),
            pltpu.SemaphoreType.DMA,
            pltpu.SemaphoreType.DMA,
        ],
    )
    return fn(sim, prompt2, yn)


def _assemble_body(big_in_ref, head_ref, part_ref, big_ref, rs_ref):
    del big_in_ref
    big_ref[...] = head_ref[...]

    @pl.when(pl.program_id(0) == 0)
    def _():
        rs_ref[...] = jnp.sum(part_ref[...]).reshape(1, 1) / float(_B)


def _run_assemble(big0, head3, part):
    return pl.pallas_call(
        _assemble_body,
        grid=(_B,),
        in_specs=[
            pl.BlockSpec(memory_space=pl.ANY),
            pl.BlockSpec((1, _HEAD, _D), lambda b: (b, 0, 0)),
            pl.BlockSpec((_NW, 16), lambda b: (0, 0)),
        ],
        out_specs=[
            pl.BlockSpec((1, _HEAD, _D), lambda b: (b, 0, 0)),
            pl.BlockSpec((1, 1), lambda b: (0, 0)),
        ],
        out_shape=[
            jax.ShapeDtypeStruct((_B, _OUT_S, _D), jnp.float32),
            jax.ShapeDtypeStruct((1, 1), jnp.float32),
        ],
        input_output_aliases={0: 0},
    )(big0, head3, part)


def kernel(x_embed, prompt, prompt_key):
    big0, mean3 = _run_mean_copy(x_embed)
    sim, yn = _run_dist(mean3.reshape(_B, _D), prompt_key)
    head, idx, bkn, part = _run_topk_gather(
        sim, prompt.reshape(_P, _ROW_W), yn)
    big, rs = _run_assemble(big0, head.reshape(_B, _HEAD, _D), part)
    return big, rs[0, 0], sim, idx, bkn


# speedup vs baseline: 1.3805x; 1.1270x over previous
import functools

import jax
import jax.numpy as jnp
from jax import lax
from jax.experimental import pallas as pl
from jax.experimental.pallas import tpu as pltpu
from jax.experimental.pallas import tpu_sc as plsc

_SCALE = 0.1
_K = 8
_L = 5
_P = 1024
_D = 768
_B = 128
_S = 196
_OUT_S = _K * _L + _S
_HEAD = _K * _L
_ROW_W = _L * _D

_NC = 2
_NS = 16
_NW = _NC * _NS
_ROWS_PER_W = _B // _NW
_CHUNKS = _P // 16


_BT = 8


def _mean_copy_body(x_ref, big_ref, mean_ref):
    xb = x_ref[...]
    big_ref[:, :_HEAD, :] = jnp.zeros((_BT, _HEAD, _D), jnp.float32)
    big_ref[:, _HEAD:, :] = xb
    mean_ref[...] = jnp.mean(xb, axis=1, keepdims=True)


def _run_mean_copy(x_embed):
    return pl.pallas_call(
        _mean_copy_body,
        grid=(_B // _BT,),
        in_specs=[pl.BlockSpec((_BT, _S, _D), lambda i: (i, 0, 0))],
        out_specs=[
            pl.BlockSpec((_BT, _OUT_S, _D), lambda i: (i, 0, 0)),
            pl.BlockSpec((_BT, 1, _D), lambda i: (i, 0, 0)),
        ],
        out_shape=[
            jax.ShapeDtypeStruct((_B, _OUT_S, _D), jnp.float32),
            jax.ShapeDtypeStruct((_B, 1, _D), jnp.float32),
        ],
    )(x_embed)


def _map_to_ball(u):
    ss = jnp.sum(u * u, axis=-1, keepdims=True)
    un = u * lax.rsqrt(jnp.maximum(ss, 1e-12))
    us = un * _SCALE
    n2 = jnp.sum(us * us, axis=-1, keepdims=True)
    n = jnp.sqrt(jnp.maximum(n2, 1e-15))
    y = jnp.tanh(n) * us / n
    yn2 = jnp.sum(y * y, axis=-1, keepdims=True)
    ynorm = jnp.sqrt(jnp.maximum(yn2, 1e-15))
    maxnorm = 1.0 - 4e-3
    return jnp.where(ynorm > maxnorm, y / ynorm * maxnorm, y)


def _dist_body(mean_ref, pk_ref, sim_ref, yn_ref):
    x = _map_to_ball(mean_ref[...])
    y = _map_to_ball(pk_ref[...])
    yn_ref[...] = y
    x2 = jnp.sum(x * x, axis=-1, keepdims=True)
    y2 = jnp.sum(y * y, axis=-1)[None, :]
    xy = lax.dot_general(x, y, (((1,), (1,)), ((), ())),
                         precision=lax.Precision.HIGHEST,
                         preferred_element_type=jnp.float32)
    alpha = 1.0 - 2.0 * xy + y2
    beta = 1.0 - x2
    num2 = alpha * alpha * x2 + beta * beta * y2 - 2.0 * alpha * beta * xy
    den = 1.0 - 2.0 * xy + x2 * y2
    norm = jnp.sqrt(jnp.maximum(num2, 1e-15)) / (den + 1e-15)
    arg = jnp.clip(norm, 0.0, 1.0 - 1e-7)
    dist = jnp.log((1.0 + arg) / (1.0 - arg))
    sim_ref[...] = -dist


def _run_dist(mean2, prompt_key):
    return pl.pallas_call(
        _dist_body,
        out_shape=[
            jax.ShapeDtypeStruct((_B, _P), jnp.float32),
            jax.ShapeDtypeStruct((_P, _D), jnp.float32),
        ],
    )(mean2, prompt_key)


def _vgather16(v, i):
    dn = lax.GatherDimensionNumbers(offset_dims=(), collapsed_slice_dims=(0,),
                                    start_index_map=(0,))
    return lax.gather(v, i[:, None], dimension_numbers=dn, slice_sizes=(1,),
                      mode=lax.GatherScatterMode.PROMISE_IN_BOUNDS)


def _sc_body(sim_hbm, prompt_hbm, key_hbm,
             head_hbm, idx_hbm, bkn_hbm, part_hbm,
             sim_v, idx128_v, idx40_v, rows_v, bkn_v, sum_v, sem, sem2):
    cid = lax.axis_index("c")
    sid = lax.axis_index("s")
    wid = cid * _NS + sid
    lane = lax.iota(jnp.int32, 16)
    neg = jnp.full((16,), -3.0e38, jnp.float32)
    zidx = jnp.zeros((16,), jnp.int32)
    zf = jnp.zeros((16,), jnp.float32)

    def row_body(r, acc):
        b = wid * _ROWS_PER_W + r
        pltpu.sync_copy(sim_hbm.at[b], sim_v)

        def chunk_body(i, carry):
            bk, bi = carry
            ck = sim_v[pl.ds(i * 16, 16)]
            ci = lane + i * 16
            sck, sci = plsc.sort_key_val(ck, ci, descending=True)
            rk = lax.rev(sck, (0,))
            ri = lax.rev(sci, (0,))
            mk = jnp.where(lane < 8, bk, rk)
            mi = jnp.where(lane < 8, bi, ri)
            return tuple(plsc.sort_key_val(mk, mi, descending=True))

        bk, bi = lax.fori_loop(0, _CHUNKS, chunk_body, (neg, zidx))
        iv = jnp.where(lane < 8, bi, jnp.int32(2147483647))
        fi, fv = plsc.sort_key_val(iv, bk, descending=False)
        idx128_v[pl.ds(0, 16)] = fi
        pltpu.sync_copy(idx128_v, idx_hbm.at[b])
        for c in range(3):
            j = lane + 16 * c
            q = lax.div(j, jnp.int32(_L))
            s = j - q * _L
            sel = _vgather16(fi, jnp.minimum(q, jnp.int32(15)))
            idx40_v[pl.ds(16 * c, 16)] = sel * _L + s
        pltpu.async_copy(prompt_hbm.at[idx40_v.at[pl.ds(0, _HEAD)]],
                         rows_v, sem).wait()
        pltpu.sync_copy(rows_v, head_hbm.at[b])
        pltpu.async_copy(key_hbm.at[idx128_v.at[pl.ds(0, 8)]],
                         bkn_v, sem2).wait()
        pltpu.sync_copy(bkn_v, bkn_hbm.at[b])
        sel_sim = jnp.where(lane < 8, fv, jnp.float32(0.0))
        return acc - jnp.sum(sel_sim)

    acc = lax.fori_loop(0, _ROWS_PER_W, row_body, jnp.float32(0.0))
    for c in range(8):
        sum_v[pl.ds(16 * c, 16)] = zf
    sum_v[pl.ds(0, 16)] = jnp.where(lane == 0, jnp.full((16,), acc), zf)
    pltpu.sync_copy(sum_v, part_hbm.at[wid])


def _run_topk_gather(sim, prompt_flat, yn):
    mesh = plsc.VectorSubcoreMesh(core_axis_name="c", subcore_axis_name="s",
                                  num_cores=_NC, num_subcores=_NS)
    fn = pl.kernel(
        _sc_body,
        out_type=[
            jax.ShapeDtypeStruct((_B, _HEAD, _D), jnp.float32),
            jax.ShapeDtypeStruct((_B, 128), jnp.int32),
            jax.ShapeDtypeStruct((_B, _K, _D), jnp.float32),
            jax.ShapeDtypeStruct((_NW, 128), jnp.float32),
        ],
        mesh=mesh,
        compiler_params=pltpu.CompilerParams(needs_layout_passes=False,
                                             use_tc_tiling_on_sc=True),
        scratch_types=[
            pltpu.VMEM((_P,), jnp.float32),
            pltpu.VMEM((128,), jnp.int32),
            pltpu.VMEM((48,), jnp.int32),
            pltpu.VMEM((_HEAD, _D), jnp.float32),
            pltpu.VMEM((_K, _D), jnp.float32),
            pltpu.VMEM((128,), jnp.float32),
            pltpu.SemaphoreType.DMA,
            pltpu.SemaphoreType.DMA,
        ],
    )
    return fn(sim, prompt_flat, yn)


def _assemble_body(big_in_ref, head_ref, part_ref, big_ref, rs_ref):
    del big_in_ref
    big_ref[...] = head_ref[...]

    @pl.when(pl.program_id(0) == 0)
    def _():
        rs_ref[...] = jnp.sum(part_ref[...]).reshape(1, 1) / float(_B)


def _run_assemble(big0, head3, part):
    return pl.pallas_call(
        _assemble_body,
        grid=(_B,),
        in_specs=[
            pl.BlockSpec(memory_space=pl.ANY),
            pl.BlockSpec((1, _HEAD, _D), lambda b: (b, 0, 0)),
            pl.BlockSpec((_NW, 128), lambda b: (0, 0)),
        ],
        out_specs=[
            pl.BlockSpec((1, _HEAD, _D), lambda b: (b, 0, 0)),
            pl.BlockSpec((1, 1), lambda b: (0, 0)),
        ],
        out_shape=[
            jax.ShapeDtypeStruct((_B, _OUT_S, _D), jnp.float32),
            jax.ShapeDtypeStruct((1, 1), jnp.float32),
        ],
        input_output_aliases={0: 0},
    )(big0, head3, part)


def kernel(x_embed, prompt, prompt_key):
    big0, mean3 = _run_mean_copy(x_embed)
    sim, yn = _run_dist(mean3.reshape(_B, _D), prompt_key)
    head, idx_pad, bkn, part = _run_topk_gather(
        sim, prompt.reshape(_P * _L, _D), yn)
    big, rs = _run_assemble(big0, head, part)
    return big, rs[0, 0], sim, idx_pad[:, :_K], bkn


# speedup vs baseline: 3.5478x; 2.5700x over previous
import functools

import jax
import jax.numpy as jnp
from jax import lax
from jax.experimental import pallas as pl
from jax.experimental.pallas import tpu as pltpu
from jax.experimental.pallas import tpu_sc as plsc

_SCALE = 0.1
_K = 8
_L = 5
_P = 1024
_D = 768
_B = 128
_S = 196
_OUT_S = _K * _L + _S
_HEAD = _K * _L
_ROW_W = _L * _D

_NC = 2
_NS = 16
_NW = _NC * _NS
_ROWS_PER_W = _B // _NW
_CHUNKS = _P // 16


_ST = 4


def _mean_copy_body(x_ref, big_ref, sum_ref):
    xb = x_ref[...]
    big_ref[...] = xb

    @pl.when(pl.program_id(0) == 0)
    def _():
        sum_ref[...] = jnp.zeros((_B, _D), jnp.float32)

    sum_ref[...] += jnp.sum(xb, axis=0)


def _run_mean_copy(xT):
    return pl.pallas_call(
        _mean_copy_body,
        grid=(_S // _ST,),
        in_specs=[pl.BlockSpec((_ST, _B, _D), lambda j: (j, 0, 0))],
        out_specs=[
            pl.BlockSpec((_ST, _B, _D), lambda j: (j + _HEAD // _ST, 0, 0)),
            pl.BlockSpec((_B, _D), lambda j: (0, 0)),
        ],
        out_shape=[
            jax.ShapeDtypeStruct((_OUT_S, _B, _D), jnp.float32),
            jax.ShapeDtypeStruct((_B, _D), jnp.float32),
        ],
    )(xT)


def _map_to_ball(u):
    ss = jnp.sum(u * u, axis=-1, keepdims=True)
    un = u * lax.rsqrt(jnp.maximum(ss, 1e-12))
    us = un * _SCALE
    n2 = jnp.sum(us * us, axis=-1, keepdims=True)
    n = jnp.sqrt(jnp.maximum(n2, 1e-15))
    y = jnp.tanh(n) * us / n
    yn2 = jnp.sum(y * y, axis=-1, keepdims=True)
    ynorm = jnp.sqrt(jnp.maximum(yn2, 1e-15))
    maxnorm = 1.0 - 4e-3
    return jnp.where(ynorm > maxnorm, y / ynorm * maxnorm, y)


def _dist_body(sum_ref, pk_ref, sim_ref, yn_ref):
    x = _map_to_ball(sum_ref[...] / float(_S))
    y = _map_to_ball(pk_ref[...])
    yn_ref[...] = y
    x2 = jnp.sum(x * x, axis=-1, keepdims=True)
    y2 = jnp.sum(y * y, axis=-1)[None, :]
    xy = lax.dot_general(x, y, (((1,), (1,)), ((), ())),
                         precision=lax.Precision.HIGHEST,
                         preferred_element_type=jnp.float32)
    alpha = 1.0 - 2.0 * xy + y2
    beta = 1.0 - x2
    num2 = alpha * alpha * x2 + beta * beta * y2 - 2.0 * alpha * beta * xy
    den = 1.0 - 2.0 * xy + x2 * y2
    norm = jnp.sqrt(jnp.maximum(num2, 1e-15)) / (den + 1e-15)
    arg = jnp.clip(norm, 0.0, 1.0 - 1e-7)
    dist = jnp.log((1.0 + arg) / (1.0 - arg))
    sim_ref[...] = -dist


def _run_dist(sum2, prompt_key):
    return pl.pallas_call(
        _dist_body,
        out_shape=[
            jax.ShapeDtypeStruct((_B, _P), jnp.float32),
            jax.ShapeDtypeStruct((_P, _D), jnp.float32),
        ],
    )(sum2, prompt_key)


def _vgather16(v, i):
    dn = lax.GatherDimensionNumbers(offset_dims=(), collapsed_slice_dims=(0,),
                                    start_index_map=(0,))
    return lax.gather(v, i[:, None], dimension_numbers=dn, slice_sizes=(1,),
                      mode=lax.GatherScatterMode.PROMISE_IN_BOUNDS)


def _sc_body(sim_hbm, prompt_hbm, key_hbm,
             head_hbm, idx_hbm, bkn_hbm, part_hbm,
             sim_v, idx128_v, idx40_v, rows_v, bkn_v, sum_v, sem, sem2):
    cid = lax.axis_index("c")
    sid = lax.axis_index("s")
    wid = cid * _NS + sid
    lane = lax.iota(jnp.int32, 16)
    neg = jnp.full((16,), -3.0e38, jnp.float32)
    zidx = jnp.zeros((16,), jnp.int32)
    zf = jnp.zeros((16,), jnp.float32)

    def row_body(r, acc):
        b = wid * _ROWS_PER_W + r
        pltpu.sync_copy(sim_hbm.at[b], sim_v)

        def chunk_body(i, carry):
            bk, bi = carry
            ck = sim_v[pl.ds(i * 16, 16)]
            ci = lane + i * 16
            sck, sci = plsc.sort_key_val(ck, ci, descending=True)
            rk = lax.rev(sck, (0,))
            ri = lax.rev(sci, (0,))
            mk = jnp.where(lane < 8, bk, rk)
            mi = jnp.where(lane < 8, bi, ri)
            return tuple(plsc.sort_key_val(mk, mi, descending=True))

        bk, bi = lax.fori_loop(0, _CHUNKS, chunk_body, (neg, zidx))
        iv = jnp.where(lane < 8, bi, jnp.int32(2147483647))
        fi, fv = plsc.sort_key_val(iv, bk, descending=False)
        idx128_v[pl.ds(0, 16)] = fi
        pltpu.sync_copy(idx128_v, idx_hbm.at[b])
        for c in range(3):
            j = lane + 16 * c
            q = lax.div(j, jnp.int32(_L))
            s = j - q * _L
            sel = _vgather16(fi, jnp.minimum(q, jnp.int32(15)))
            idx40_v[pl.ds(16 * c, 16)] = s * _P + sel
        pltpu.async_copy(prompt_hbm.at[idx40_v.at[pl.ds(0, _HEAD)]],
                         rows_v, sem).wait()
        pltpu.sync_copy(rows_v, head_hbm.at[b])
        pltpu.async_copy(key_hbm.at[idx128_v.at[pl.ds(0, 8)]],
                         bkn_v, sem2).wait()
        pltpu.sync_copy(bkn_v, bkn_hbm.at[b])
        sel_sim = jnp.where(lane < 8, fv, jnp.float32(0.0))
        return acc - jnp.sum(sel_sim)

    acc = lax.fori_loop(0, _ROWS_PER_W, row_body, jnp.float32(0.0))
    for c in range(8):
        sum_v[pl.ds(16 * c, 16)] = zf
    sum_v[pl.ds(0, 16)] = jnp.where(lane == 0, jnp.full((16,), acc), zf)
    pltpu.sync_copy(sum_v, part_hbm.at[wid])


def _run_topk_gather(sim, prompt_flat, yn):
    mesh = plsc.VectorSubcoreMesh(core_axis_name="c", subcore_axis_name="s",
                                  num_cores=_NC, num_subcores=_NS)
    fn = pl.kernel(
        _sc_body,
        out_type=[
            jax.ShapeDtypeStruct((_B, _HEAD, _D), jnp.float32),
            jax.ShapeDtypeStruct((_B, 128), jnp.int32),
            jax.ShapeDtypeStruct((_B, _K, _D), jnp.float32),
            jax.ShapeDtypeStruct((_NW, 128), jnp.float32),
        ],
        mesh=mesh,
        compiler_params=pltpu.CompilerParams(needs_layout_passes=False,
                                             use_tc_tiling_on_sc=True),
        scratch_types=[
            pltpu.VMEM((_P,), jnp.float32),
            pltpu.VMEM((128,), jnp.int32),
            pltpu.VMEM((48,), jnp.int32),
            pltpu.VMEM((_HEAD, _D), jnp.float32),
            pltpu.VMEM((_K, _D), jnp.float32),
            pltpu.VMEM((128,), jnp.float32),
            pltpu.SemaphoreType.DMA,
            pltpu.SemaphoreType.DMA,
        ],
    )
    return fn(sim, prompt_flat, yn)


_DBT = 8


def _assemble_body(big_in_ref, head_ref, part_ref, big_ref, rs_ref):
    del big_in_ref
    big_ref[...] = jnp.transpose(head_ref[...], (1, 0, 2))

    @pl.when(pl.program_id(0) == 0)
    def _():
        rs_ref[...] = jnp.sum(part_ref[...]).reshape(1, 1) / float(_B)


def _run_assemble(bigT0, head, part):
    return pl.pallas_call(
        _assemble_body,
        grid=(_B // _DBT,),
        in_specs=[
            pl.BlockSpec(memory_space=pl.ANY),
            pl.BlockSpec((_DBT, _HEAD, _D), lambda b: (b, 0, 0)),
            pl.BlockSpec((_NW, 128), lambda b: (0, 0)),
        ],
        out_specs=[
            pl.BlockSpec((_HEAD, _DBT, _D), lambda b: (0, b, 0)),
            pl.BlockSpec((1, 1), lambda b: (0, 0)),
        ],
        out_shape=[
            jax.ShapeDtypeStruct((_OUT_S, _B, _D), jnp.float32),
            jax.ShapeDtypeStruct((1, 1), jnp.float32),
        ],
        input_output_aliases={0: 0},
    )(bigT0, head, part)


def kernel(x_embed, prompt, prompt_key):
    xT = jnp.transpose(x_embed, (1, 0, 2))
    bigT0, sum2 = _run_mean_copy(xT)
    sim, yn = _run_dist(sum2, prompt_key)
    prompt_flat = jnp.transpose(prompt, (1, 0, 2)).reshape(_L * _P, _D)
    head, idx_pad, bkn, part = _run_topk_gather(sim, prompt_flat, yn)
    bigT, rs = _run_assemble(bigT0, head, part)
    big = jnp.transpose(bigT, (1, 0, 2))
    return big, rs[0, 0], sim, idx_pad[:, :_K], bkn


# speedup vs baseline: 3.6528x; 1.0296x over previous
import functools

import jax
import jax.numpy as jnp
from jax import lax
from jax.experimental import pallas as pl
from jax.experimental.pallas import tpu as pltpu
from jax.experimental.pallas import tpu_sc as plsc

_SCALE = 0.1
_K = 8
_L = 5
_P = 1024
_D = 768
_B = 128
_S = 196
_OUT_S = _K * _L + _S
_HEAD = _K * _L
_ROW_W = _L * _D

_NC = 2
_NS = 16
_NW = _NC * _NS
_ROWS_PER_W = _B // _NW
_CHUNKS = _P // 16


_ST = 4


def _map_to_ball(u):
    ss = jnp.sum(u * u, axis=-1, keepdims=True)
    un = u * lax.rsqrt(jnp.maximum(ss, 1e-12))
    us = un * _SCALE
    n2 = jnp.sum(us * us, axis=-1, keepdims=True)
    n = jnp.sqrt(jnp.maximum(n2, 1e-15))
    y = jnp.tanh(n) * us / n
    yn2 = jnp.sum(y * y, axis=-1, keepdims=True)
    ynorm = jnp.sqrt(jnp.maximum(yn2, 1e-15))
    maxnorm = 1.0 - 4e-3
    return jnp.where(ynorm > maxnorm, y / ynorm * maxnorm, y)


def _mean_copy_dist_body(x_ref, pk_ref, big_ref, sim_ref, yn_ref, sum_ref):
    j = pl.program_id(0)
    xb = x_ref[...]
    big_ref[...] = xb

    @pl.when(j == 0)
    def _():
        sum_ref[...] = jnp.zeros((_B, _D), jnp.float32)

    sum_ref[...] += jnp.sum(xb, axis=0)

    @pl.when(j == _S // _ST - 1)
    def _():
        x = _map_to_ball(sum_ref[...] / float(_S))
        y = _map_to_ball(pk_ref[...])
        yn_ref[...] = y
        x2 = jnp.sum(x * x, axis=-1, keepdims=True)
        y2 = jnp.sum(y * y, axis=-1)[None, :]
        xy = lax.dot_general(x, y, (((1,), (1,)), ((), ())),
                             precision=lax.Precision.HIGHEST,
                             preferred_element_type=jnp.float32)
        alpha = 1.0 - 2.0 * xy + y2
        beta = 1.0 - x2
        num2 = alpha * alpha * x2 + beta * beta * y2 - 2.0 * alpha * beta * xy
        den = 1.0 - 2.0 * xy + x2 * y2
        norm = jnp.sqrt(jnp.maximum(num2, 1e-15)) / (den + 1e-15)
        arg = jnp.clip(norm, 0.0, 1.0 - 1e-7)
        dist = jnp.log((1.0 + arg) / (1.0 - arg))
        sim_ref[...] = -dist


def _run_mean_copy_dist(xT, prompt_key):
    return pl.pallas_call(
        _mean_copy_dist_body,
        grid=(_S // _ST,),
        in_specs=[
            pl.BlockSpec((_ST, _B, _D), lambda j: (j, 0, 0)),
            pl.BlockSpec((_P, _D), lambda j: (0, 0)),
        ],
        out_specs=[
            pl.BlockSpec((_ST, _B, _D), lambda j: (j + _HEAD // _ST, 0, 0)),
            pl.BlockSpec((_B, _P), lambda j: (0, 0)),
            pl.BlockSpec((_P, _D), lambda j: (0, 0)),
        ],
        out_shape=[
            jax.ShapeDtypeStruct((_OUT_S, _B, _D), jnp.float32),
            jax.ShapeDtypeStruct((_B, _P), jnp.float32),
            jax.ShapeDtypeStruct((_P, _D), jnp.float32),
        ],
        scratch_shapes=[pltpu.VMEM((_B, _D), jnp.float32)],
    )(xT, prompt_key)


def _vgather16(v, i):
    dn = lax.GatherDimensionNumbers(offset_dims=(), collapsed_slice_dims=(0,),
                                    start_index_map=(0,))
    return lax.gather(v, i[:, None], dimension_numbers=dn, slice_sizes=(1,),
                      mode=lax.GatherScatterMode.PROMISE_IN_BOUNDS)


def _sc_body(sim_hbm, prompt_hbm, key_hbm,
             head_hbm, idx_hbm, bkn_hbm, part_hbm,
             sim_v, idx128_v, idx40_v, rows_v, bkn_v, sum_v, sem, sem2):
    cid = lax.axis_index("c")
    sid = lax.axis_index("s")
    wid = cid * _NS + sid
    lane = lax.iota(jnp.int32, 16)
    neg = jnp.full((16,), -3.0e38, jnp.float32)
    zidx = jnp.zeros((16,), jnp.int32)
    zf = jnp.zeros((16,), jnp.float32)

    def row_body(r, acc):
        b = wid * _ROWS_PER_W + r
        pltpu.sync_copy(sim_hbm.at[b], sim_v)

        def chunk_body(i, carry):
            bk, bi = carry
            ck = sim_v[pl.ds(i * 16, 16)]
            ci = lane + i * 16
            sck, sci = plsc.sort_key_val(ck, ci, descending=True)
            rk = lax.rev(sck, (0,))
            ri = lax.rev(sci, (0,))
            mk = jnp.where(lane < 8, bk, rk)
            mi = jnp.where(lane < 8, bi, ri)
            return tuple(plsc.sort_key_val(mk, mi, descending=True))

        bk, bi = lax.fori_loop(0, _CHUNKS, chunk_body, (neg, zidx))
        iv = jnp.where(lane < 8, bi, jnp.int32(2147483647))
        fi, fv = plsc.sort_key_val(iv, bk, descending=False)
        idx128_v[pl.ds(0, 16)] = fi
        pltpu.sync_copy(idx128_v, idx_hbm.at[b])
        for c in range(3):
            j = lane + 16 * c
            q = lax.div(j, jnp.int32(_L))
            s = j - q * _L
            sel = _vgather16(fi, jnp.minimum(q, jnp.int32(15)))
            idx40_v[pl.ds(16 * c, 16)] = s * _P + sel
        pltpu.async_copy(prompt_hbm.at[idx40_v.at[pl.ds(0, _HEAD)]],
                         rows_v, sem).wait()
        pltpu.sync_copy(rows_v, head_hbm.at[b])
        pltpu.async_copy(key_hbm.at[idx128_v.at[pl.ds(0, 8)]],
                         bkn_v, sem2).wait()
        pltpu.sync_copy(bkn_v, bkn_hbm.at[b])
        sel_sim = jnp.where(lane < 8, fv, jnp.float32(0.0))
        return acc - jnp.sum(sel_sim)

    acc = lax.fori_loop(0, _ROWS_PER_W, row_body, jnp.float32(0.0))
    for c in range(8):
        sum_v[pl.ds(16 * c, 16)] = zf
    sum_v[pl.ds(0, 16)] = jnp.where(lane == 0, jnp.full((16,), acc), zf)
    pltpu.sync_copy(sum_v, part_hbm.at[wid])


def _run_topk_gather(sim, prompt_flat, yn):
    mesh = plsc.VectorSubcoreMesh(core_axis_name="c", subcore_axis_name="s",
                                  num_cores=_NC, num_subcores=_NS)
    fn = pl.kernel(
        _sc_body,
        out_type=[
            jax.ShapeDtypeStruct((_B, _HEAD, _D), jnp.float32),
            jax.ShapeDtypeStruct((_B, 128), jnp.int32),
            jax.ShapeDtypeStruct((_B, _K, _D), jnp.float32),
            jax.ShapeDtypeStruct((_NW, 128), jnp.float32),
        ],
        mesh=mesh,
        compiler_params=pltpu.CompilerParams(needs_layout_passes=False,
                                             use_tc_tiling_on_sc=True),
        scratch_types=[
            pltpu.VMEM((_P,), jnp.float32),
            pltpu.VMEM((128,), jnp.int32),
            pltpu.VMEM((48,), jnp.int32),
            pltpu.VMEM((_HEAD, _D), jnp.float32),
            pltpu.VMEM((_K, _D), jnp.float32),
            pltpu.VMEM((128,), jnp.float32),
            pltpu.SemaphoreType.DMA,
            pltpu.SemaphoreType.DMA,
        ],
    )
    return fn(sim, prompt_flat, yn)


_DBT = 16


def _assemble_body(big_in_ref, head_ref, part_ref, big_ref, rs_ref):
    del big_in_ref
    big_ref[...] = jnp.transpose(head_ref[...], (1, 0, 2))

    @pl.when(pl.program_id(0) == 0)
    def _():
        rs_ref[...] = jnp.sum(part_ref[...]).reshape(1, 1) / float(_B)


def _run_assemble(bigT0, head, part):
    return pl.pallas_call(
        _assemble_body,
        grid=(_B // _DBT,),
        in_specs=[
            pl.BlockSpec(memory_space=pl.ANY),
            pl.BlockSpec((_DBT, _HEAD, _D), lambda b: (b, 0, 0)),
            pl.BlockSpec((_NW, 128), lambda b: (0, 0)),
        ],
        out_specs=[
            pl.BlockSpec((_HEAD, _DBT, _D), lambda b: (0, b, 0)),
            pl.BlockSpec((1, 1), lambda b: (0, 0)),
        ],
        out_shape=[
            jax.ShapeDtypeStruct((_OUT_S, _B, _D), jnp.float32),
            jax.ShapeDtypeStruct((1, 1), jnp.float32),
        ],
        input_output_aliases={0: 0},
    )(bigT0, head, part)


def kernel(x_embed, prompt, prompt_key):
    xT = jnp.transpose(x_embed, (1, 0, 2))
    bigT0, sim, yn = _run_mean_copy_dist(xT, prompt_key)
    prompt_flat = jnp.transpose(prompt, (1, 0, 2)).reshape(_L * _P, _D)
    head, idx_pad, bkn, part = _run_topk_gather(sim, prompt_flat, yn)
    bigT, rs = _run_assemble(bigT0, head, part)
    big = jnp.transpose(bigT, (1, 0, 2))
    return big, rs[0, 0], sim, idx_pad[:, :_K], bkn
